# baseline (speedup 1.0000x reference)
import jax
import jax.numpy as jnp
from jax.experimental import pallas as pl
from jax.experimental.pallas import tpu as pltpu

B, P, DIN, D = 16, 4096, 128, 128
N_MASK = P // 2
LN_EPS = 1e-5
SPB = 4


def _fused_kernel(x_ref, noise_ref, pos_enc_ref, pos_pred_ref, w_emb_ref,
                  w_enc_ref, w_pred_ref, b_emb_ref, b_enc_ref, mtok_ref,
                  b_pred_ref, out_ref, mask_ref, mrow_s, mt_s, pecat_s,
                  pp2_s, wcat_s, mv_s):
    f32 = jnp.float32
    pid = pl.program_id(0)

    @pl.when(pid == 0)
    def _prologue():
        noise = noise_ref[...]
        bits = jax.lax.bitcast_convert_type(noise, jnp.int32)
        idxrev = (P - 1) - jax.lax.broadcasted_iota(jnp.int32, (B, P), 1)

        active = jnp.ones((B, P), dtype=jnp.bool_)
        k = jnp.full((B, 1), N_MASK, dtype=jnp.int32)
        tv = jnp.zeros((B, 1), dtype=jnp.int32)
        tir = jnp.zeros((B, 1), dtype=jnp.int32)

        passes = [(0, 24), (0, 18), (0, 12), (0, 6), (0, 0), (1, 6), (1, 0)]
        for src, shift in passes:
            dsrc = bits if src == 0 else idxrev
            d = jax.lax.shift_right_logical(dsrc, shift) & 63
            dm = jnp.where(active, d, -1)
            s = jnp.zeros((B, 1), dtype=jnp.int32)
            for m in (32, 16, 8, 4, 2, 1):
                cand = s + m
                cnt = jnp.sum((dm >= cand).astype(jnp.int32),
                              axis=1, keepdims=True)
                s = jnp.where(cnt >= k, cand, s)
            cnt_gt = jnp.sum((dm > s).astype(jnp.int32),
                             axis=1, keepdims=True)
            k = k - cnt_gt
            active = dm == s
            if src == 0:
                tv = tv | jax.lax.shift_left(s, shift)
            else:
                tir = tir | jax.lax.shift_left(s, shift)

        masked = (bits > tv) | ((bits == tv) & (idxrev >= tir))
        mask_ref[...] = masked
        mrow_s[...] = masked.astype(f32)
        mt_s[...] = masked.astype(jnp.bfloat16).T

        pecat_s[...] = (
            jnp.dot(pos_enc_ref[...] + b_emb_ref[...], w_enc_ref[...],
                    preferred_element_type=f32) + b_enc_ref[...])
        pp2_s[...] = jnp.dot(pos_pred_ref[...], w_pred_ref[...],
                             preferred_element_type=f32)
        wcat_s[...] = jnp.dot(w_emb_ref[...], w_enc_ref[...],
                              preferred_element_type=f32)
        mv_s[...] = (jnp.dot(mtok_ref[...], w_pred_ref[...],
                             preferred_element_type=f32) + b_pred_ref[...])

    f32 = jnp.float32
    bf16 = jnp.bfloat16
    o_mat = jnp.full((D, D), 1.0 / D, dtype=bf16)
    for j in range(SPB):
        sid = pid * SPB + j
        xb = x_ref[j]
        c = jnp.dot(xb, wcat_s[...], preferred_element_type=f32) + pecat_s[...]
        mu = jnp.dot(c.astype(bf16), o_mat, preferred_element_type=f32)
        cm = c - mu
        var = jnp.dot((cm * cm).astype(bf16), o_mat,
                      preferred_element_type=f32)
        ln = cm * jax.lax.rsqrt(var + LN_EPS)
        vrow = (1.0 - mrow_s[pl.ds(sid, 1), :]).astype(bf16)
        s = jnp.dot(vrow, ln.astype(bf16), preferred_element_type=f32)
        ctx_mean = s * (1.0 / (P - N_MASK))
        v = jnp.dot(ctx_mean, w_pred_ref[...],
                    preferred_element_type=f32) + mv_s[...]
        subiota = jax.lax.broadcasted_iota(jnp.int32, (B, D), 0)
        onehot = jnp.where(subiota == sid, 1.0, 0.0).astype(bf16)
        mfull = jnp.dot(mt_s[...], onehot, preferred_element_type=f32)
        out_ref[j] = c + mfull * (pp2_s[...] + v - c)


def kernel(x, W_emb, b_emb, pos_emb_enc, W_enc, b_enc, mask_token,
           pos_emb_pred, W_pred, b_pred, mask_noise):
    f32 = jnp.float32
    b_emb2 = b_emb.reshape(1, D)
    b_enc2 = b_enc.reshape(1, D)
    mtok2 = mask_token.reshape(1, D)
    b_pred2 = b_pred.reshape(1, D)

    const = lambda i: (0, 0)
    x_like, maskI = pl.pallas_call(
        _fused_kernel,
        grid=(B // SPB,),
        in_specs=[
            pl.BlockSpec((SPB, P, DIN), lambda i: (i, 0, 0)),
            pl.BlockSpec((B, P), const),
            pl.BlockSpec((P, D), const),
            pl.BlockSpec((P, D), const),
            pl.BlockSpec((DIN, D), const),
            pl.BlockSpec((D, D), const),
            pl.BlockSpec((D, D), const),
            pl.BlockSpec((1, D), const),
            pl.BlockSpec((1, D), const),
            pl.BlockSpec((1, D), const),
            pl.BlockSpec((1, D), const),
        ],
        out_specs=(
            pl.BlockSpec((SPB, P, D), lambda i: (i, 0, 0)),
            pl.BlockSpec((B, P), const),
        ),
        out_shape=(
            jax.ShapeDtypeStruct((B, P, D), f32),
            jax.ShapeDtypeStruct((B, P), jnp.bool_),
        ),
        scratch_shapes=[
            pltpu.VMEM((B, P), f32),
            pltpu.VMEM((P, B), jnp.bfloat16),
            pltpu.VMEM((P, D), f32),
            pltpu.VMEM((P, D), f32),
            pltpu.VMEM((DIN, D), f32),
            pltpu.VMEM((1, D), f32),
        ],
    )(x, mask_noise, pos_emb_enc, pos_emb_pred, W_emb, W_enc, W_pred,
      b_emb2, b_enc2, mtok2, b_pred2)

    return x_like, maskI


# speedup vs baseline: 1.0022x; 1.0022x over previous
import jax
import jax.numpy as jnp
from jax.experimental import pallas as pl
from jax.experimental.pallas import tpu as pltpu

B, P, DIN, D = 16, 4096, 128, 128
N_MASK = P // 2
LN_EPS = 1e-5
SPB = 4


def _fused_kernel(x_ref, noise_ref, pos_enc_ref, pos_pred_ref, w_emb_ref,
                  w_enc_ref, w_pred_ref, b_emb_ref, b_enc_ref, mtok_ref,
                  b_pred_ref, out_ref, mask_ref, mrow_s, mt_s, pe2_s,
                  pp2_s, wc_s, mv_s):
    f32 = jnp.float32
    pid = pl.program_id(0)

    @pl.when(pid == 0)
    def _prologue():
        noise = noise_ref[...]
        bits = jax.lax.bitcast_convert_type(noise, jnp.int32)
        idxrev = (P - 1) - jax.lax.broadcasted_iota(jnp.int32, (B, P), 1)

        active = jnp.ones((B, P), dtype=jnp.bool_)
        k = jnp.full((B, 1), N_MASK, dtype=jnp.int32)
        tv = jnp.zeros((B, 1), dtype=jnp.int32)
        tir = jnp.zeros((B, 1), dtype=jnp.int32)

        passes = [(0, 24), (0, 18), (0, 12), (0, 6), (0, 0), (1, 6), (1, 0)]
        for src, shift in passes:
            dsrc = bits if src == 0 else idxrev
            d = jax.lax.shift_right_logical(dsrc, shift) & 63
            dm = jnp.where(active, d, -1)
            s = jnp.zeros((B, 1), dtype=jnp.int32)
            for m in (32, 16, 8, 4, 2, 1):
                cand = s + m
                cnt = jnp.sum((dm >= cand).astype(jnp.int32),
                              axis=1, keepdims=True)
                s = jnp.where(cnt >= k, cand, s)
            cnt_gt = jnp.sum((dm > s).astype(jnp.int32),
                             axis=1, keepdims=True)
            k = k - cnt_gt
            active = dm == s
            if src == 0:
                tv = tv | jax.lax.shift_left(s, shift)
            else:
                tir = tir | jax.lax.shift_left(s, shift)

        masked = (bits > tv) | ((bits == tv) & (idxrev >= tir))
        mask_ref[...] = masked
        mrow_s[...] = masked.astype(f32)
        mt_s[...] = masked.astype(jnp.bfloat16).T

        pe2_s[...] = (
            jnp.dot(pos_enc_ref[...] + b_emb_ref[...], w_enc_ref[...],
                    preferred_element_type=f32) + b_enc_ref[...])
        pp2_s[...] = jnp.dot(pos_pred_ref[...], w_pred_ref[...],
                             preferred_element_type=f32)
        wc_s[...] = jnp.dot(w_emb_ref[...], w_enc_ref[...],
                              preferred_element_type=f32)
        mv_s[...] = (jnp.dot(mtok_ref[...], w_pred_ref[...],
                             preferred_element_type=f32) + b_pred_ref[...])

    f32 = jnp.float32
    bf16 = jnp.bfloat16
    o_mat = jnp.full((D, D), 1.0 / D, dtype=bf16)
    for j in range(SPB):
        sid = pid * SPB + j
        xb = x_ref[j]
        c = jnp.dot(xb, wc_s[...], preferred_element_type=f32) + pe2_s[...]
        mu = jnp.dot(c.astype(bf16), o_mat, preferred_element_type=f32)
        cm = c - mu
        var = jnp.dot((cm * cm).astype(bf16), o_mat,
                      preferred_element_type=f32)
        ln = cm * jax.lax.rsqrt(var + LN_EPS)
        vrow = (1.0 - mrow_s[pl.ds(sid, 1), :]).astype(bf16)
        s = jnp.dot(vrow, ln.astype(bf16), preferred_element_type=f32)
        ctx_mean = s * (1.0 / (P - N_MASK))
        v = jnp.dot(ctx_mean, w_pred_ref[...],
                    preferred_element_type=f32) + mv_s[...]
        subiota = jax.lax.broadcasted_iota(jnp.int32, (B, D), 0)
        onehot = jnp.where(subiota == sid, 1.0, 0.0).astype(bf16)
        mfull = jnp.dot(mt_s[...], onehot, preferred_element_type=f32)
        out_ref[j] = c + mfull * (pp2_s[...] + v - c)


def kernel(x, W_emb, b_emb, pos_emb_enc, W_enc, b_enc, mask_token,
           pos_emb_pred, W_pred, b_pred, mask_noise):
    f32 = jnp.float32
    b_emb2 = b_emb.reshape(1, D)
    b_enc2 = b_enc.reshape(1, D)
    mtok2 = mask_token.reshape(1, D)
    b_pred2 = b_pred.reshape(1, D)

    const = lambda i: (0, 0)
    x_like, maskI = pl.pallas_call(
        _fused_kernel,
        grid=(B // SPB,),
        in_specs=[
            pl.BlockSpec((SPB, P, DIN), lambda i: (i, 0, 0)),
            pl.BlockSpec((B, P), const),
            pl.BlockSpec((P, D), const),
            pl.BlockSpec((P, D), const),
            pl.BlockSpec((DIN, D), const),
            pl.BlockSpec((D, D), const),
            pl.BlockSpec((D, D), const),
            pl.BlockSpec((1, D), const),
            pl.BlockSpec((1, D), const),
            pl.BlockSpec((1, D), const),
            pl.BlockSpec((1, D), const),
        ],
        out_specs=(
            pl.BlockSpec((SPB, P, D), lambda i: (i, 0, 0)),
            pl.BlockSpec((B, P), const),
        ),
        out_shape=(
            jax.ShapeDtypeStruct((B, P, D), f32),
            jax.ShapeDtypeStruct((B, P), jnp.bool_),
        ),
        scratch_shapes=[
            pltpu.VMEM((B, P), f32),
            pltpu.VMEM((P, B), jnp.bfloat16),
            pltpu.VMEM((P, D), f32),
            pltpu.VMEM((P, D), f32),
            pltpu.VMEM((DIN, D), f32),
            pltpu.VMEM((1, D), f32),
        ],
    )(x, mask_noise, pos_emb_enc, pos_emb_pred, W_emb, W_enc, W_pred,
      b_emb2, b_enc2, mtok2, b_pred2)

    return x_like, maskI
